# Initial kernel scaffold; baseline (speedup 1.0000x reference)
import jax
import jax.numpy as jnp
from jax.experimental import pallas as pl


def kernel(states, edges, type_weights, type_biases, gru_kernel, gru_rec_kernel, gru_bias):
    raise NotImplementedError("write your pallas kernel here")


# speedup vs baseline: 10.6360x; 10.6360x over previous
import functools

import jax
import jax.numpy as jnp
from jax import lax
from jax.experimental import pallas as pl
from jax.experimental.pallas import tpu as pltpu
from jax.experimental.pallas import tpu_sc as plsc

_N = 10000
_E = 320000
_H = 128
_T = 4
_STEPS = [3, 1]

_NC = 2
_NS = 16
_NW = _NC * _NS
_EW = _E // _NW
_C = 80
_NCHUNK = _EW // _C
_NP = 10240
_RPT = _NP // _NS


def _row_block(i):
    return (i, 0)


def _tc_table_body(s_ref, w_ref, b_ref, o_ref):
    o_ref[...] = (
        jnp.dot(s_ref[...], w_ref[...], preferred_element_type=jnp.float32)
        + b_ref[...]
    )


def _tc_table(states, wcat, bcat, blk):
    grid = (_N // blk,)
    return pl.pallas_call(
        _tc_table_body,
        grid=grid,
        in_specs=[
            pl.BlockSpec((blk, _H), _row_block),
            pl.BlockSpec((_H, _T * _H), lambda i: (0, 0)),
            pl.BlockSpec((1, _T * _H), lambda i: (0, 0)),
        ],
        out_specs=pl.BlockSpec((blk, _T * _H), _row_block),
        out_shape=jax.ShapeDtypeStruct((_N, _T * _H), jnp.float32),
    )(states, wcat, bcat)


def _tc_gru_body(p_ref, h_ref, k_ref, r_ref, b_ref, o_ref):
    x = p_ref[0] + p_ref[1]
    h = h_ref[...]
    mx = jnp.dot(x, k_ref[...], preferred_element_type=jnp.float32) + b_ref[0:1, :]
    mi = jnp.dot(h, r_ref[...], preferred_element_type=jnp.float32) + b_ref[1:2, :]
    z = jax.nn.sigmoid(mx[:, :_H] + mi[:, :_H])
    r = jax.nn.sigmoid(mx[:, _H:2 * _H] + mi[:, _H:2 * _H])
    hh = jnp.tanh(mx[:, 2 * _H:] + r * mi[:, 2 * _H:])
    o_ref[...] = z * h + (1.0 - z) * hh


def _tc_gru(parts, h, gk, grk, gb, blk):
    grid = (_N // blk,)
    return pl.pallas_call(
        _tc_gru_body,
        grid=grid,
        in_specs=[
            pl.BlockSpec((2, blk, _H), lambda i: (0, i, 0)),
            pl.BlockSpec((blk, _H), _row_block),
            pl.BlockSpec((_H, 3 * _H), lambda i: (0, 0)),
            pl.BlockSpec((_H, 3 * _H), lambda i: (0, 0)),
            pl.BlockSpec((2, 3 * _H), lambda i: (0, 0)),
        ],
        out_specs=pl.BlockSpec((blk, _H), _row_block),
        out_shape=jax.ShapeDtypeStruct((_N, _H), jnp.float32),
    )(parts, h, gk, grk, gb)


def _sc_edge_body(yb_hbm, key_hbm, dst_hbm, zeros_hbm, out_hbm,
                  key_v, dst_v, rows_v, sem, acc_sh):
    cid = lax.axis_index("c")
    sid = lax.axis_index("s")
    zbase = sid * _RPT
    pltpu.sync_copy(zeros_hbm.at[pl.ds(zbase, _RPT)], acc_sh.at[pl.ds(zbase, _RPT)])
    plsc.subcore_barrier()

    wid = cid * _NS + sid
    ebase = wid * _EW

    def body(k, carry):
        off = pl.multiple_of(ebase + k * _C, _C)
        pltpu.sync_copy(key_hbm.at[pl.ds(off, _C)], key_v)
        pltpu.async_copy(yb_hbm.at[key_v], rows_v, sem).wait()
        pltpu.sync_copy(dst_hbm.at[pl.ds(off, _C)], dst_v)
        pltpu.sync_copy(rows_v, acc_sh.at[dst_v], add=True)
        return carry

    lax.fori_loop(0, _NCHUNK, body, 0)
    plsc.subcore_barrier()
    obase = cid * _NP + sid * _RPT
    pltpu.sync_copy(acc_sh.at[pl.ds(zbase, _RPT)], out_hbm.at[pl.ds(obase, _RPT)])


@functools.partial(
    pl.kernel,
    out_type=jax.ShapeDtypeStruct((_NC * _NP, _H), jnp.float32),
    mesh=plsc.VectorSubcoreMesh(core_axis_name="c", subcore_axis_name="s"),
    scratch_types=[
        pltpu.VMEM((_C,), jnp.int32),
        pltpu.VMEM((_C,), jnp.int32),
        pltpu.VMEM((_C, _H), jnp.float32),
        pltpu.SemaphoreType.DMA,
        pltpu.VMEM_SHARED((_NP, _H), jnp.float32),
    ],
)
def _sc_edge(yb, key, dst, zeros, out, key_v, dst_v, rows_v, sem, acc_sh):
    _sc_edge_body(yb, key, dst, zeros, out, key_v, dst_v, rows_v, sem, acc_sh)


def kernel(states, edges, type_weights, type_biases, gru_kernel,
           gru_rec_kernel, gru_bias):
    etype = edges[:, 0].astype(jnp.int32)
    src = edges[:, 1].astype(jnp.int32)
    dst = edges[:, 2].astype(jnp.int32)
    key = src * _T + etype
    zeros = jnp.zeros((_NP, _H), jnp.float32)

    h = states
    for layer, steps in enumerate(_STEPS):
        wcat = jnp.transpose(type_weights[layer], (1, 0, 2)).reshape(_H, _T * _H)
        bcat = type_biases[layer].reshape(1, _T * _H)
        gk = gru_kernel[layer]
        grk = gru_rec_kernel[layer]
        gb = gru_bias[layer]
        for _ in range(steps):
            yb = _tc_table(h, wcat, bcat, 1000)
            yb_flat = yb.reshape(_N * _T, _H)
            parts = _sc_edge(yb_flat, key, dst, zeros)
            h = _tc_gru(parts.reshape(_NC, _NP, _H), h, gk, grk, gb, 1000)
    return h


# speedup vs baseline: 26.0545x; 2.4497x over previous
import functools

import jax
import jax.numpy as jnp
from jax import lax
from jax.experimental import pallas as pl
from jax.experimental.pallas import tpu as pltpu
from jax.experimental.pallas import tpu_sc as plsc

_N = 10000
_E = 320000
_H = 128
_T = 4
_STEPS = [3, 1]

_NC = 2
_NS = 16
_NW = _NC * _NS
_EW = _E // _NW
_C = 40
_NCHUNK = _EW // _C
_NP = 10240
_RPT = _NP // _NS


def _row_block(i):
    return (i, 0)


def _tc_table_body(s_ref, w_ref, b_ref, o_ref):
    o_ref[...] = (
        jnp.dot(s_ref[...], w_ref[...], preferred_element_type=jnp.float32)
        + b_ref[...]
    )


def _tc_table(states, wcat, bcat, blk):
    grid = (_N // blk,)
    return pl.pallas_call(
        _tc_table_body,
        grid=grid,
        in_specs=[
            pl.BlockSpec((blk, _H), _row_block),
            pl.BlockSpec((_H, _T * _H), lambda i: (0, 0)),
            pl.BlockSpec((1, _T * _H), lambda i: (0, 0)),
        ],
        out_specs=pl.BlockSpec((blk, _T * _H), _row_block),
        out_shape=jax.ShapeDtypeStruct((_N, _T * _H), jnp.float32),
    )(states, wcat, bcat)


def _tc_gru_body(p_ref, h_ref, k_ref, r_ref, b_ref, o_ref):
    x = p_ref[0] + p_ref[1]
    h = h_ref[...]
    mx = jnp.dot(x, k_ref[...], preferred_element_type=jnp.float32) + b_ref[0:1, :]
    mi = jnp.dot(h, r_ref[...], preferred_element_type=jnp.float32) + b_ref[1:2, :]
    z = jax.nn.sigmoid(mx[:, :_H] + mi[:, :_H])
    r = jax.nn.sigmoid(mx[:, _H:2 * _H] + mi[:, _H:2 * _H])
    hh = jnp.tanh(mx[:, 2 * _H:] + r * mi[:, 2 * _H:])
    o_ref[...] = z * h + (1.0 - z) * hh


def _tc_gru(parts, h, gk, grk, gb, blk):
    grid = (_N // blk,)
    return pl.pallas_call(
        _tc_gru_body,
        grid=grid,
        in_specs=[
            pl.BlockSpec((2, blk, _H), lambda i: (0, i, 0)),
            pl.BlockSpec((blk, _H), _row_block),
            pl.BlockSpec((_H, 3 * _H), lambda i: (0, 0)),
            pl.BlockSpec((_H, 3 * _H), lambda i: (0, 0)),
            pl.BlockSpec((2, 3 * _H), lambda i: (0, 0)),
        ],
        out_specs=pl.BlockSpec((blk, _H), _row_block),
        out_shape=jax.ShapeDtypeStruct((_N, _H), jnp.float32),
    )(parts, h, gk, grk, gb)


_R = 5
_NG = _NCHUNK // _R


def _sc_edge_body(yb_hbm, key_hbm, dst_hbm, zeros_hbm, out_hbm,
                  key_v, dsts, rows, gsems, dsems, acc_sh):
    cid = lax.axis_index("c")
    sid = lax.axis_index("s")
    zbase = sid * _RPT
    pltpu.sync_copy(zeros_hbm.at[pl.ds(zbase, _RPT)], acc_sh.at[pl.ds(zbase, _RPT)])

    wid = cid * _NS + sid
    ebase = wid * _EW
    pltpu.sync_copy(key_hbm.at[pl.ds(ebase, _EW)], key_v)
    plsc.subcore_barrier()

    def issue(k, j):
        off = pl.multiple_of(k * _C, 8)
        pltpu.async_copy(dst_hbm.at[pl.ds(ebase + off, _C)], dsts[j], dsems[j])
        pltpu.async_copy(yb_hbm.at[key_v.at[pl.ds(off, _C)]], rows[j], gsems[j])

    for j in range(_R):
        issue(j, j)

    def grp(gi, carry):
        for j in range(_R):
            k = gi * _R + j
            pltpu.make_async_copy(dst_hbm.at[pl.ds(0, _C)], dsts[j], dsems[j]).wait()
            pltpu.make_async_copy(yb_hbm.at[pl.ds(0, _C)], rows[j], gsems[j]).wait()
            pltpu.sync_copy(rows[j], acc_sh.at[dsts[j]], add=True)

            @pl.when(k + _R < _NCHUNK)
            def _():
                issue(k + _R, j)
        return carry

    lax.fori_loop(0, _NG, grp, 0)
    plsc.subcore_barrier()
    obase = cid * _NP + sid * _RPT
    pltpu.sync_copy(acc_sh.at[pl.ds(zbase, _RPT)], out_hbm.at[pl.ds(obase, _RPT)])


@functools.partial(
    pl.kernel,
    out_type=jax.ShapeDtypeStruct((_NC * _NP, _H), jnp.float32),
    mesh=plsc.VectorSubcoreMesh(core_axis_name="c", subcore_axis_name="s"),
    scratch_types=[
        pltpu.VMEM((_EW,), jnp.int32),
        [pltpu.VMEM((_C,), jnp.int32)] * _R,
        [pltpu.VMEM((_C, _H), jnp.float32)] * _R,
        [pltpu.SemaphoreType.DMA] * _R,
        [pltpu.SemaphoreType.DMA] * _R,
        pltpu.VMEM_SHARED((_NP, _H), jnp.float32),
    ],
)
def _sc_edge(yb, key, dst, zeros, out, key_v, dsts, rows, gsems, dsems, acc_sh):
    _sc_edge_body(yb, key, dst, zeros, out, key_v, dsts, rows, gsems, dsems, acc_sh)


def kernel(states, edges, type_weights, type_biases, gru_kernel,
           gru_rec_kernel, gru_bias):
    etype = edges[:, 0].astype(jnp.int32)
    src = edges[:, 1].astype(jnp.int32)
    dst = edges[:, 2].astype(jnp.int32)
    key = src * _T + etype
    zeros = jnp.zeros((_NP, _H), jnp.float32)

    h = states
    for layer, steps in enumerate(_STEPS):
        wcat = jnp.transpose(type_weights[layer], (1, 0, 2)).reshape(_H, _T * _H)
        bcat = type_biases[layer].reshape(1, _T * _H)
        gk = gru_kernel[layer]
        grk = gru_rec_kernel[layer]
        gb = gru_bias[layer]
        for _ in range(steps):
            yb = _tc_table(h, wcat, bcat, 1000)
            yb_flat = yb.reshape(_N * _T, _H)
            parts = _sc_edge(yb_flat, key, dst, zeros)
            h = _tc_gru(parts.reshape(_NC, _NP, _H), h, gk, grk, gb, 1000)
    return h
